# baseline (speedup 1.0000x reference)
import functools

import jax
import jax.numpy as jnp
from jax import lax
from jax.experimental import pallas as pl
from jax.experimental.pallas import tpu as pltpu
from jax.experimental.pallas import tpu_sc as plsc

_N = 10000
_E = 160000
_EP = _E + _N
_C = 128
_LAT = 64
_LOW = 32
_CLU = 10
_SIG = 0.5
_H = 10
_HID = 128
_NP = 10240
_NT = 32
_RT = _NP // _NT
_CAP = 24576
_G = 128
_S = 2000
_BLK = 256
_NB = _NP // _BLK


def _mesh():
    return plsc.VectorSubcoreMesh(core_axis_name="c", subcore_axis_name="s")


_SC_PARAMS = pltpu.CompilerParams(needs_layout_passes=False)


def _wid():
    return lax.axis_index("c") * 16 + lax.axis_index("s")


@functools.partial(
    pl.kernel,
    mesh=_mesh(),
    compiler_params=_SC_PARAMS,
    out_type=(
        jax.ShapeDtypeStruct((_NT * _CAP,), jnp.int32),
        jax.ShapeDtypeStruct((_NT * _CAP,), jnp.int32),
        jax.ShapeDtypeStruct((_NT * 16,), jnp.int32),
    ),
    scratch_types=[
        pltpu.VMEM((_S,), jnp.int32),
        pltpu.VMEM((_S,), jnp.int32),
        pltpu.VMEM((_CAP + 16,), jnp.int32),
        pltpu.VMEM((_CAP + 16,), jnp.int32),
        pltpu.VMEM((16,), jnp.int32),
        pltpu.SemaphoreType.DMA,
    ],
)
def _sc_partition(src_hbm, dst_hbm, src_c, dst_c, counts, sv, dv, sbuf, dbuf,
                  cbuf, sem):
    wid = _wid()
    base = wid * _RT
    zero = jnp.zeros((16,), jnp.int32)
    basev = jnp.full((16,), base, jnp.int32)
    lanes = lax.iota(jnp.int32, 16)
    dumpv = jnp.full((16,), _CAP, jnp.int32) + lanes

    def _zero_body(i, _):
        sbuf[pl.ds(pl.multiple_of(i * 16, 16), 16)] = zero
        dbuf[pl.ds(pl.multiple_of(i * 16, 16), 16)] = basev
        return 0

    lax.fori_loop(0, (_CAP + 16) // 16, _zero_body, 0)

    def _chunk(ci, w):
        off = pl.multiple_of(ci * _S, 8)
        pltpu.sync_copy(src_hbm.at[pl.ds(off, _S)], sv)
        pltpu.sync_copy(dst_hbm.at[pl.ds(off, _S)], dv)

        def _grp(g, w):
            o = pl.multiple_of(g * 16, 16)
            d16 = dv[pl.ds(o, 16)]
            s16 = sv[pl.ds(o, 16)]
            m = (d16 >= base) & (d16 < base + _RT)
            cs = plsc.cumsum(m.astype(jnp.int32))
            wc = jnp.minimum(w, _CAP - 16)
            pos = jnp.where(m, cs + (wc - 1), dumpv)
            plsc.store_scatter(sbuf, [pos], s16)
            plsc.store_scatter(dbuf, [pos], d16)
            return w + cs[15]

        return lax.fori_loop(0, _S // 16, _grp, w)

    w = lax.fori_loop(0, _EP // _S, _chunk, jnp.int32(0))
    cbuf[...] = jnp.full((16,), w, jnp.int32)
    obase = pl.multiple_of(wid * _CAP, 8)
    pltpu.sync_copy(sbuf.at[pl.ds(0, _CAP)], src_c.at[pl.ds(obase, _CAP)])
    pltpu.sync_copy(dbuf.at[pl.ds(0, _CAP)], dst_c.at[pl.ds(obase, _CAP)])
    pltpu.sync_copy(cbuf, counts.at[pl.ds(pl.multiple_of(wid * 16, 8), 16)])


def _make_sce(hpr, G, nb):
    npass = _H // hpr
    nvs = (128 // hpr) // 16

    @functools.partial(
        pl.kernel,
        mesh=_mesh(),
        compiler_params=_SC_PARAMS,
        out_type=jax.ShapeDtypeStruct((npass * _NP, 128), jnp.float32),
        scratch_types=[
            pltpu.VMEM((_RT, 128), jnp.float32),
            pltpu.VMEM((_RT, 16), jnp.float32),
            [pltpu.VMEM((G, 128), jnp.float32) for _ in range(nb)],
            pltpu.VMEM((G,), jnp.int32),
            [pltpu.VMEM((G,), jnp.int32) for _ in range(nb)],
            [pltpu.VMEM((G,), jnp.int32) for _ in range(nb)],
            [[pltpu.VMEM((G,), jnp.float32) for _ in range(hpr)]
             for _ in range(nb)],
            [pltpu.VMEM((_N,), jnp.float32) for _ in range(hpr)],
            [pltpu.VMEM((_N,), jnp.float32) for _ in range(hpr)],
            pltpu.VMEM((16,), jnp.int32),
            [pltpu.SemaphoreType.DMA for _ in range(nb)],
        ],
    )
    def _sce(xw_hbm, asrc_hbm, adst_hbm, srcc_hbm, dstc_hbm, cnt_hbm, agg_hbm,
             acc, accd, rowss, sidx, didxs, aidxs, pbufss, at_ss, at_ds,
             cntv, sems):
        wid = _wid()
        base = wid * _RT
        pltpu.sync_copy(cnt_hbm.at[pl.ds(pl.multiple_of(wid * 16, 8), 16)], cntv)
        cnt = jnp.minimum(cntv[pl.ds(0, 16)][0], _CAP)
        nch = (cnt + G - 1) // G
        zf = jnp.zeros((16,), jnp.float32)
        lanes = lax.iota(jnp.int32, 16)

        def _pass(q, _):
            for sub in range(hpr):
                toff = pl.multiple_of((q * hpr + sub) * _NP, 8)
                pltpu.sync_copy(asrc_hbm.at[pl.ds(toff, _N)], at_ss[sub])
                pltpu.sync_copy(adst_hbm.at[pl.ds(toff, _N)], at_ds[sub])

            @plsc.parallel_loop(0, _RT, 1, unroll=4)
            def _zr(r):
                for c in range(8):
                    acc[r, pl.ds(16 * c, 16)] = zf
                accd[r, pl.ds(0, 16)] = zf

            hb = jnp.full((16,), q * _NP, jnp.int32)

            def _stage(ci, b):
                off = pl.multiple_of(wid * _CAP + ci * G, 8)
                pltpu.sync_copy(srcc_hbm.at[pl.ds(off, G)], sidx)
                pltpu.sync_copy(dstc_hbm.at[pl.ds(off, G)], didxs[b])

                @plsc.parallel_loop(0, G // 16, 1, unroll=2)
                def _pgrp(g):
                    o = pl.multiple_of(g * 16, 16)
                    s16 = sidx[pl.ds(o, 16)]
                    d16 = didxs[b][pl.ds(o, 16)]
                    aidxs[b][pl.ds(o, 16)] = s16 + hb
                    valid = (lanes + o) < (cnt - ci * G)
                    for sub in range(hpr):
                        av = (plsc.load_gather(at_ss[sub], [s16])
                              + plsc.load_gather(at_ds[sub], [d16]))
                        av = jnp.where(av > 0, av, 0.2 * av)
                        pbufss[b][sub][pl.ds(o, 16)] = jnp.where(
                            valid, jnp.exp(av), 0.0)

                pltpu.async_copy(xw_hbm.at[aidxs[b]], rowss[b], sems[b])

            def _proc(b):
                pltpu.make_async_copy(
                    xw_hbm.at[pl.ds(0, G)], rowss[b], sems[b]).wait()

                @plsc.parallel_loop(0, G // 16, 1, unroll=2)
                def _egrp(g):
                    o = pl.multiple_of(g * 16, 16)
                    dv16 = jnp.clip(didxs[b][pl.ds(o, 16)] - base, 0, _RT - 1)
                    pv16s = [pbufss[b][sub][pl.ds(o, 16)]
                             for sub in range(hpr)]
                    for lane in range(16):
                        dl = dv16[lane]
                        pvs = [jnp.full((16,), pv16s[sub][lane], jnp.float32)
                               for sub in range(hpr)]
                        for c in range(8):
                            plsc.addupdate(
                                acc.at[dl, pl.ds(16 * c, 16)],
                                pvs[c // nvs] * rowss[b][o + lane,
                                                         pl.ds(16 * c, 16)])
                        if hpr == 1:
                            pvc = pvs[0]
                        else:
                            pvc = jnp.where(lanes < 8, pvs[0], pvs[1])
                        plsc.addupdate(accd.at[dl], pvc)

            if nb == 1:
                def _chunk(ci, _):
                    _stage(ci, 0)
                    _proc(0)
                    return 0

                lax.fori_loop(0, nch, _chunk, 0)
            else:
                @pl.when(nch > 0)
                def _():
                    _stage(0, 0)

                def _outer(co, _):
                    for b in range(2):
                        ci = co * 2 + b

                        @pl.when(ci < nch)
                        def _():
                            @pl.when(ci + 1 < nch)
                            def _():
                                _stage(ci + 1, 1 - b)

                            _proc(b)
                    return 0

                lax.fori_loop(0, (nch + 1) // 2, _outer, 0)

            @plsc.parallel_loop(0, _RT, 1, unroll=4)
            def _nrm(r):
                invv = 1.0 / (accd[r, pl.ds(0, 16)] + 1e-16)
                ivs = [jnp.full((16,), invv[8 * sub], jnp.float32)
                       for sub in range(hpr)]
                for c in range(8):
                    acc[r, pl.ds(16 * c, 16)] = (acc[r, pl.ds(16 * c, 16)]
                                                 * ivs[c // nvs])
            pltpu.sync_copy(
                acc,
                agg_hbm.at[pl.ds(pl.multiple_of(q * _NP + base, 8), _RT)])
            return 0

        lax.fori_loop(0, npass, _pass, 0)

    return _sce


_sce_l1 = _make_sce(1, 96, 2)
_sce_l2 = _make_sce(2, 48, 1)


def _tca_body(x_ref, w1r_ref, wasT_ref, wadT_ref, xw_ref, as_ref, ad_ref):
    xb = x_ref[...]
    for h in range(_H):
        xw_ref[h] = jnp.dot(xb, w1r_ref[h], preferred_element_type=jnp.float32)
    dn = (((1,), (1,)), ((), ()))
    as_ref[...] = lax.dot_general(wasT_ref[...], xb, dn,
                                  preferred_element_type=jnp.float32)
    ad_ref[...] = lax.dot_general(wadT_ref[...], xb, dn,
                                  preferred_element_type=jnp.float32)


def _tca(xp, w1r, wasT, wadT):
    return pl.pallas_call(
        _tca_body,
        grid=(_NB,),
        in_specs=[
            pl.BlockSpec((_BLK, _C), lambda b: (b, 0)),
            pl.BlockSpec((_H, _C, _HID), lambda b: (0, 0, 0)),
            pl.BlockSpec((16, _C), lambda b: (0, 0)),
            pl.BlockSpec((16, _C), lambda b: (0, 0)),
        ],
        out_specs=[
            pl.BlockSpec((_H, _BLK, _HID), lambda b: (0, b, 0)),
            pl.BlockSpec((16, _BLK), lambda b: (0, b)),
            pl.BlockSpec((16, _BLK), lambda b: (0, b)),
        ],
        out_shape=[
            jax.ShapeDtypeStruct((_H, _NP, _HID), jnp.float32),
            jax.ShapeDtypeStruct((16, _NP), jnp.float32),
            jax.ShapeDtypeStruct((16, _NP), jnp.float32),
        ],
    )(xp, w1r, wasT, wadT)


def _tcb_body(agg1_ref, w2r_ref, b1r_ref, as2_ref, ad2_ref,
              xw2_ref, la_ref, ld_ref):
    h1 = [jax.nn.relu(agg1_ref[hi] + b1r_ref[hi]) for hi in range(_H)]
    for ho in range(_H):
        acc = jnp.zeros((_BLK, _LAT), jnp.float32)
        for hi in range(_H):
            acc = acc + jnp.dot(h1[hi], w2r_ref[hi, :, ho, :],
                                preferred_element_type=jnp.float32)
        xw2_ref[ho // 2, :, pl.ds((ho % 2) * _LAT, _LAT)] = acc
        dn = (((1,), (1,)), ((), ()))
        la_ref[ho] = lax.dot_general(as2_ref[ho], acc, dn,
                                     preferred_element_type=jnp.float32)
        ld_ref[ho] = lax.dot_general(ad2_ref[ho], acc, dn,
                                     preferred_element_type=jnp.float32)


def _tcb(agg1, w2r, b1r, as2, ad2):
    return pl.pallas_call(
        _tcb_body,
        grid=(_NB,),
        in_specs=[
            pl.BlockSpec((_H, _BLK, _HID), lambda b: (0, b, 0)),
            pl.BlockSpec((_H, _HID, _H, _LAT), lambda b: (0, 0, 0, 0)),
            pl.BlockSpec((_H, 1, _HID), lambda b: (0, 0, 0)),
            pl.BlockSpec((_H, 1, _LAT), lambda b: (0, 0, 0)),
            pl.BlockSpec((_H, 1, _LAT), lambda b: (0, 0, 0)),
        ],
        out_specs=[
            pl.BlockSpec((_H // 2, _BLK, 128), lambda b: (0, b, 0)),
            pl.BlockSpec((_H, 1, _BLK), lambda b: (0, 0, b)),
            pl.BlockSpec((_H, 1, _BLK), lambda b: (0, 0, b)),
        ],
        out_shape=[
            jax.ShapeDtypeStruct((_H // 2, _NP, 128), jnp.float32),
            jax.ShapeDtypeStruct((_H, 1, _NP), jnp.float32),
            jax.ShapeDtypeStruct((_H, 1, _NP), jnp.float32),
        ],
    )(agg1, w2r, b1r, as2, ad2)


def _tcc_body(agg2_ref, b2_ref, wf_ref, bf_ref, wc_ref, bc_ref,
              gxn_ref, gxc_ref, cs_ref):
    b = pl.program_id(0)
    gx = jnp.zeros((_BLK, _LAT), jnp.float32)
    for q in range(_H // 2):
        gx = gx + agg2_ref[q, :, : _LAT] + agg2_ref[q, :, _LAT:]
    gx = jax.nn.relu(gx * (1.0 / _H) + b2_ref[...])
    gxn_ref[...] = jnp.dot(gx, wf_ref[...],
                           preferred_element_type=jnp.float32) + bf_ref[...]
    gxc_ref[...] = jnp.dot(gx, wc_ref[...],
                           preferred_element_type=jnp.float32) + bc_ref[...]
    rid = lax.broadcasted_iota(jnp.int32, (_BLK, 1), 0) + b * _BLK
    gxm = jnp.where(rid < _N, gx, 0.0)

    @pl.when(b == 0)
    def _():
        cs_ref[...] = jnp.zeros_like(cs_ref)

    cs_ref[...] += gxm


def _tcc(agg2, b2, wf, bf, wc16, bc16):
    return pl.pallas_call(
        _tcc_body,
        grid=(_NB,),
        in_specs=[
            pl.BlockSpec((_H // 2, _BLK, 128), lambda b: (0, b, 0)),
            pl.BlockSpec((1, _LAT), lambda b: (0, 0)),
            pl.BlockSpec((_LAT, _LOW), lambda b: (0, 0)),
            pl.BlockSpec((1, _LOW), lambda b: (0, 0)),
            pl.BlockSpec((_LAT, 16), lambda b: (0, 0)),
            pl.BlockSpec((1, 16), lambda b: (0, 0)),
        ],
        out_specs=[
            pl.BlockSpec((_BLK, _LOW), lambda b: (b, 0)),
            pl.BlockSpec((_BLK, 16), lambda b: (b, 0)),
            pl.BlockSpec((_BLK, _LAT), lambda b: (0, 0)),
        ],
        out_shape=[
            jax.ShapeDtypeStruct((_NP, _LOW), jnp.float32),
            jax.ShapeDtypeStruct((_NP, 16), jnp.float32),
            jax.ShapeDtypeStruct((_BLK, _LAT), jnp.float32),
        ],
    )(agg2, b2, wf, bf, wc16, bc16)


def _tcm_body(x_ref, wm1_ref, bm1_ref, wm2_ref, bm2_ref, wf_ref, bf_ref,
              gxn_ref, cs_ref):
    b = pl.program_id(0)
    t = jax.nn.relu(jnp.dot(x_ref[...], wm1_ref[...],
                            preferred_element_type=jnp.float32) + bm1_ref[...])
    gx2 = jnp.dot(t, wm2_ref[...],
                  preferred_element_type=jnp.float32) + bm2_ref[...]
    gxn_ref[...] = jnp.dot(gx2, wf_ref[...],
                           preferred_element_type=jnp.float32) + bf_ref[...]
    rid = lax.broadcasted_iota(jnp.int32, (_BLK, 1), 0) + b * _BLK
    gxm = jnp.where(rid < _N, gx2, 0.0)

    @pl.when(b == 0)
    def _():
        cs_ref[...] = jnp.zeros_like(cs_ref)

    cs_ref[...] += gxm


def _tcm(xp, wm1, bm1, wm2, bm2, wf, bf):
    return pl.pallas_call(
        _tcm_body,
        grid=(_NB,),
        in_specs=[
            pl.BlockSpec((_BLK, _C), lambda b: (b, 0)),
            pl.BlockSpec((_C, _LAT), lambda b: (0, 0)),
            pl.BlockSpec((1, _LAT), lambda b: (0, 0)),
            pl.BlockSpec((_LAT, _LAT), lambda b: (0, 0)),
            pl.BlockSpec((1, _LAT), lambda b: (0, 0)),
            pl.BlockSpec((_LAT, _LOW), lambda b: (0, 0)),
            pl.BlockSpec((1, _LOW), lambda b: (0, 0)),
        ],
        out_specs=[
            pl.BlockSpec((_BLK, _LOW), lambda b: (b, 0)),
            pl.BlockSpec((_BLK, _LAT), lambda b: (0, 0)),
        ],
        out_shape=[
            jax.ShapeDtypeStruct((_NP, _LOW), jnp.float32),
            jax.ShapeDtypeStruct((_BLK, _LAT), jnp.float32),
        ],
    )(xp, wm1, bm1, wm2, bm2, wf, bf)


def kernel(x, edge_index, W1, att_src1, att_dst1, b1, W2, att_src2, att_dst2,
           b2, Wm1, bm1, Wm2, bm2, Wf, bf, Wc, bc):
    loop = jnp.arange(_N, dtype=edge_index.dtype)
    src = jnp.concatenate([edge_index[0], loop])
    dst = jnp.concatenate([edge_index[1], loop])

    noise = jax.random.normal(jax.random.key(42), x.shape, x.dtype) * _SIG
    nrm = jnp.linalg.norm(noise, axis=1, keepdims=True)
    x_aug = x + noise / jnp.maximum(nrm, 1e-12)
    xp = jnp.pad(x, ((0, _NP - _N), (0, 0)))
    xap = jnp.pad(x_aug, ((0, _NP - _N), (0, 0)))

    w1r = W1.reshape(_C, _H, _HID).transpose(1, 0, 2)
    wasT = jnp.pad(jnp.einsum('hcf,hf->hc', w1r, att_src1), ((0, 6), (0, 0)))
    wadT = jnp.pad(jnp.einsum('hcf,hf->hc', w1r, att_dst1), ((0, 6), (0, 0)))
    w2r = W2.reshape(_H, _HID, _H, _LAT)
    b1r = b1.reshape(_H, 1, _HID)
    as2 = att_src2.reshape(_H, 1, _LAT)
    ad2 = att_dst2.reshape(_H, 1, _LAT)
    b2r = b2.reshape(1, _LAT)
    bfr = bf.reshape(1, _LOW)
    wc16 = jnp.pad(Wc, ((0, 0), (0, 16 - _CLU)))
    bc16 = jnp.pad(bc, (0, 16 - _CLU)).reshape(1, 16)
    bm1r = bm1.reshape(1, _LAT)
    bm2r = bm2.reshape(1, _LAT)

    src_c, dst_c, counts = _sc_partition(src, dst)

    def enc(hp):
        xw1, as1, ad1 = _tca(hp, w1r, wasT, wadT)
        agg1 = _sce_l1(xw1.reshape(_H * _NP, _HID), as1.reshape(-1),
                       ad1.reshape(-1), src_c, dst_c, counts)
        xw2, la2, ld2 = _tcb(agg1.reshape(_H, _NP, _HID), w2r, b1r, as2, ad2)
        agg2 = _sce_l2(xw2.reshape(_H // 2 * _NP, 128), la2.reshape(-1),
                       ld2.reshape(-1), src_c, dst_c, counts)
        return _tcc(agg2.reshape(_H // 2, _NP, 128), b2r, Wf, bfr, wc16, bc16)

    gxn0, gxc0, cs0 = enc(xp)
    gxn1, gxc1, cs1 = enc(xap)
    gxn2, cs2 = _tcm(xp, Wm1, bm1r, Wm2, bm2r, Wf, bfr)

    g0 = jnp.sum(cs0, axis=0) / _N
    g1 = jnp.sum(cs1, axis=0) / _N
    g2 = jnp.sum(cs2, axis=0) / _N
    fenzi = jnp.exp(jnp.dot(g0, g1) / 0.2)
    fenmu = (fenzi + jnp.exp(jnp.dot(g0, g2) / 0.2)
             + jnp.exp(jnp.dot(g1, g2) / 0.2))
    loss_graph = -jnp.log10(fenzi / fenmu)

    return (gxn0[:_N], gxn1[:_N], gxn2[:_N], loss_graph,
            gxc0[:_N, :_CLU].T, gxc1[:_N, :_CLU].T)


# speedup vs baseline: 1.2117x; 1.2117x over previous
import functools

import jax
import jax.numpy as jnp
from jax import lax
from jax.experimental import pallas as pl
from jax.experimental.pallas import tpu as pltpu
from jax.experimental.pallas import tpu_sc as plsc

_N = 10000
_E = 160000
_EP = _E + _N
_C = 128
_LAT = 64
_LOW = 32
_CLU = 10
_SIG = 0.5
_H = 10
_HID = 128
_NP = 10240
_NT = 32
_RT = _NP // _NT
_CAP = 24576
_G = 128
_S = 2000
_BLK = 256
_NB = _NP // _BLK


def _mesh():
    return plsc.VectorSubcoreMesh(core_axis_name="c", subcore_axis_name="s")


_SC_PARAMS = pltpu.CompilerParams(needs_layout_passes=False)


def _wid():
    return lax.axis_index("c") * 16 + lax.axis_index("s")


@functools.partial(
    pl.kernel,
    mesh=_mesh(),
    compiler_params=_SC_PARAMS,
    out_type=(
        jax.ShapeDtypeStruct((_NT * _CAP,), jnp.int32),
        jax.ShapeDtypeStruct((_NT * _CAP,), jnp.int32),
        jax.ShapeDtypeStruct((_NT * 16,), jnp.int32),
    ),
    scratch_types=[
        pltpu.VMEM((_S,), jnp.int32),
        pltpu.VMEM((_S,), jnp.int32),
        pltpu.VMEM((_CAP + 16,), jnp.int32),
        pltpu.VMEM((_CAP + 16,), jnp.int32),
        pltpu.VMEM((16,), jnp.int32),
        pltpu.SemaphoreType.DMA,
    ],
)
def _sc_partition(src_hbm, dst_hbm, src_c, dst_c, counts, sv, dv, sbuf, dbuf,
                  cbuf, sem):
    wid = _wid()
    base = wid * _RT
    zero = jnp.zeros((16,), jnp.int32)
    basev = jnp.full((16,), base, jnp.int32)
    lanes = lax.iota(jnp.int32, 16)
    dumpv = jnp.full((16,), _CAP, jnp.int32) + lanes

    def _zero_body(i, _):
        sbuf[pl.ds(pl.multiple_of(i * 16, 16), 16)] = zero
        dbuf[pl.ds(pl.multiple_of(i * 16, 16), 16)] = basev
        return 0

    lax.fori_loop(0, (_CAP + 16) // 16, _zero_body, 0)

    def _chunk(ci, w):
        off = pl.multiple_of(ci * _S, 8)
        pltpu.sync_copy(src_hbm.at[pl.ds(off, _S)], sv)
        pltpu.sync_copy(dst_hbm.at[pl.ds(off, _S)], dv)

        def _grp(g, w):
            o = pl.multiple_of(g * 16, 16)
            d16 = dv[pl.ds(o, 16)]
            s16 = sv[pl.ds(o, 16)]
            m = (d16 >= base) & (d16 < base + _RT)
            cs = plsc.cumsum(m.astype(jnp.int32))
            wc = jnp.minimum(w, _CAP - 16)
            pos = jnp.where(m, cs + (wc - 1), dumpv)
            plsc.store_scatter(sbuf, [pos], s16)
            plsc.store_scatter(dbuf, [pos], d16)
            return w + cs[15]

        return lax.fori_loop(0, _S // 16, _grp, w)

    w = lax.fori_loop(0, _EP // _S, _chunk, jnp.int32(0))
    cbuf[...] = jnp.full((16,), w, jnp.int32)
    obase = pl.multiple_of(wid * _CAP, 8)
    pltpu.sync_copy(sbuf.at[pl.ds(0, _CAP)], src_c.at[pl.ds(obase, _CAP)])
    pltpu.sync_copy(dbuf.at[pl.ds(0, _CAP)], dst_c.at[pl.ds(obase, _CAP)])
    pltpu.sync_copy(cbuf, counts.at[pl.ds(pl.multiple_of(wid * 16, 8), 16)])


_P = 1024


@functools.partial(
    pl.kernel,
    mesh=_mesh(),
    compiler_params=_SC_PARAMS,
    out_type=jax.ShapeDtypeStruct((_H * _NT * _CAP,), jnp.float32),
    scratch_types=[
        pltpu.VMEM((_N,), jnp.float32),
        pltpu.VMEM((_N,), jnp.float32),
        pltpu.VMEM((_P,), jnp.int32),
        pltpu.VMEM((_P,), jnp.int32),
        pltpu.VMEM((_P,), jnp.float32),
        pltpu.VMEM((16,), jnp.int32),
    ],
)
def _sc_logits(asrc_hbm, adst_hbm, srcc_hbm, dstc_hbm, cnt_hbm, p_hbm,
               at_s, at_d, si, di, po, cntv):
    wid = _wid()
    pltpu.sync_copy(cnt_hbm.at[pl.ds(pl.multiple_of(wid * 16, 8), 16)], cntv)
    cnt = jnp.minimum(cntv[pl.ds(0, 16)][0], _CAP)
    nch = (cnt + _P - 1) // _P
    lanes = lax.iota(jnp.int32, 16)

    def _head(h, _):
        toff = pl.multiple_of(h * _NP, 8)
        pltpu.sync_copy(asrc_hbm.at[pl.ds(toff, _N)], at_s)
        pltpu.sync_copy(adst_hbm.at[pl.ds(toff, _N)], at_d)

        def _chunk(ci, _):
            off = pl.multiple_of(wid * _CAP + ci * _P, 8)
            pltpu.sync_copy(srcc_hbm.at[pl.ds(off, _P)], si)
            pltpu.sync_copy(dstc_hbm.at[pl.ds(off, _P)], di)

            @plsc.parallel_loop(0, _P // 16, 1, unroll=2)
            def _grp(g):
                o = pl.multiple_of(g * 16, 16)
                s16 = si[pl.ds(o, 16)]
                d16 = di[pl.ds(o, 16)]
                av = (plsc.load_gather(at_s, [s16])
                      + plsc.load_gather(at_d, [d16]))
                av = jnp.where(av > 0, av, 0.2 * av)
                valid = (lanes + o) < (cnt - ci * _P)
                po[pl.ds(o, 16)] = jnp.where(valid, jnp.exp(av), 0.0)

            pltpu.sync_copy(
                po,
                p_hbm.at[pl.ds(pl.multiple_of(h * _NT * _CAP + off, 8), _P)])
            return 0

        lax.fori_loop(0, nch, _chunk, 0)
        return 0

    lax.fori_loop(0, _H, _head, 0)


def _make_sce(hpr, G, nb):
    npass = _H // hpr
    nvs = (128 // hpr) // 16

    @functools.partial(
        pl.kernel,
        mesh=_mesh(),
        compiler_params=_SC_PARAMS,
        out_type=jax.ShapeDtypeStruct((npass * _NP, 128), jnp.float32),
        scratch_types=[
            pltpu.VMEM((_RT, 128), jnp.float32),
            pltpu.VMEM((_RT, 16), jnp.float32),
            [pltpu.VMEM((G, 128), jnp.float32) for _ in range(nb)],
            [pltpu.VMEM((G,), jnp.int32) for _ in range(nb)],
            [pltpu.VMEM((G,), jnp.int32) for _ in range(nb)],
            [[pltpu.VMEM((G,), jnp.float32) for _ in range(hpr)]
             for _ in range(nb)],
            pltpu.VMEM((16,), jnp.int32),
            [pltpu.SemaphoreType.DMA for _ in range(nb)],
        ],
    )
    def _sce(xw_hbm, p_hbm, srcc_hbm, dstc_hbm, cnt_hbm, agg_hbm,
             acc, accd, rowss, didxs, aidxs, pbufss, cntv, sems):
        wid = _wid()
        base = wid * _RT
        pltpu.sync_copy(cnt_hbm.at[pl.ds(pl.multiple_of(wid * 16, 8), 16)], cntv)
        cnt = jnp.minimum(cntv[pl.ds(0, 16)][0], _CAP)
        nch = (cnt + G - 1) // G
        zf = jnp.zeros((16,), jnp.float32)
        lanes = lax.iota(jnp.int32, 16)

        def _pass(q, _):
            @plsc.parallel_loop(0, _RT, 1, unroll=4)
            def _zr(r):
                for c in range(8):
                    acc[r, pl.ds(16 * c, 16)] = zf
                accd[r, pl.ds(0, 16)] = zf

            hb = jnp.full((16,), q * _NP, jnp.int32)

            def _stage(ci, b):
                off = pl.multiple_of(wid * _CAP + ci * G, 8)
                pltpu.sync_copy(srcc_hbm.at[pl.ds(off, G)], aidxs[b])
                pltpu.sync_copy(dstc_hbm.at[pl.ds(off, G)], didxs[b])
                for sub in range(hpr):
                    poff = pl.multiple_of(
                        (q * hpr + sub) * _NT * _CAP + off, 8)
                    pltpu.sync_copy(p_hbm.at[pl.ds(poff, G)],
                                    pbufss[b][sub])

                @plsc.parallel_loop(0, G // 16, 1, unroll=2)
                def _adj(g):
                    o = pl.multiple_of(g * 16, 16)
                    aidxs[b][pl.ds(o, 16)] = aidxs[b][pl.ds(o, 16)] + hb

                pltpu.async_copy(xw_hbm.at[aidxs[b]], rowss[b], sems[b])

            def _proc(b):
                pltpu.make_async_copy(
                    xw_hbm.at[pl.ds(0, G)], rowss[b], sems[b]).wait()

                @plsc.parallel_loop(0, G // 16, 1, unroll=2)
                def _egrp(g):
                    o = pl.multiple_of(g * 16, 16)
                    dv16 = jnp.clip(didxs[b][pl.ds(o, 16)] - base, 0, _RT - 1)
                    pv16s = [pbufss[b][sub][pl.ds(o, 16)]
                             for sub in range(hpr)]
                    for lane in range(16):
                        dl = dv16[lane]
                        pvs = [jnp.full((16,), pv16s[sub][lane], jnp.float32)
                               for sub in range(hpr)]
                        for c in range(8):
                            plsc.addupdate(
                                acc.at[dl, pl.ds(16 * c, 16)],
                                pvs[c // nvs] * rowss[b][o + lane,
                                                         pl.ds(16 * c, 16)])
                        if hpr == 1:
                            pvc = pvs[0]
                        else:
                            pvc = jnp.where(lanes < 8, pvs[0], pvs[1])
                        plsc.addupdate(accd.at[dl], pvc)

            if nb == 1:
                def _chunk(ci, _):
                    _stage(ci, 0)
                    _proc(0)
                    return 0

                lax.fori_loop(0, nch, _chunk, 0)
            else:
                @pl.when(nch > 0)
                def _():
                    _stage(0, 0)

                def _outer(co, _):
                    for b in range(2):
                        ci = co * 2 + b

                        @pl.when(ci < nch)
                        def _():
                            @pl.when(ci + 1 < nch)
                            def _():
                                _stage(ci + 1, 1 - b)

                            _proc(b)
                    return 0

                lax.fori_loop(0, (nch + 1) // 2, _outer, 0)

            @plsc.parallel_loop(0, _RT, 1, unroll=4)
            def _nrm(r):
                invv = 1.0 / (accd[r, pl.ds(0, 16)] + 1e-16)
                ivs = [jnp.full((16,), invv[8 * sub], jnp.float32)
                       for sub in range(hpr)]
                for c in range(8):
                    acc[r, pl.ds(16 * c, 16)] = (acc[r, pl.ds(16 * c, 16)]
                                                 * ivs[c // nvs])
            pltpu.sync_copy(
                acc,
                agg_hbm.at[pl.ds(pl.multiple_of(q * _NP + base, 8), _RT)])
            return 0

        lax.fori_loop(0, npass, _pass, 0)

    return _sce


_sce_l1 = _make_sce(1, 128, 2)
_sce_l2 = _make_sce(2, 128, 2)


def _tca_body(x_ref, w1r_ref, wasT_ref, wadT_ref, xw_ref, as_ref, ad_ref):
    xb = x_ref[...]
    for h in range(_H):
        xw_ref[h] = jnp.dot(xb, w1r_ref[h], preferred_element_type=jnp.float32)
    dn = (((1,), (1,)), ((), ()))
    as_ref[...] = lax.dot_general(wasT_ref[...], xb, dn,
                                  preferred_element_type=jnp.float32)
    ad_ref[...] = lax.dot_general(wadT_ref[...], xb, dn,
                                  preferred_element_type=jnp.float32)


def _tca(xp, w1r, wasT, wadT):
    return pl.pallas_call(
        _tca_body,
        grid=(_NB,),
        in_specs=[
            pl.BlockSpec((_BLK, _C), lambda b: (b, 0)),
            pl.BlockSpec((_H, _C, _HID), lambda b: (0, 0, 0)),
            pl.BlockSpec((16, _C), lambda b: (0, 0)),
            pl.BlockSpec((16, _C), lambda b: (0, 0)),
        ],
        out_specs=[
            pl.BlockSpec((_H, _BLK, _HID), lambda b: (0, b, 0)),
            pl.BlockSpec((16, _BLK), lambda b: (0, b)),
            pl.BlockSpec((16, _BLK), lambda b: (0, b)),
        ],
        out_shape=[
            jax.ShapeDtypeStruct((_H, _NP, _HID), jnp.float32),
            jax.ShapeDtypeStruct((16, _NP), jnp.float32),
            jax.ShapeDtypeStruct((16, _NP), jnp.float32),
        ],
    )(xp, w1r, wasT, wadT)


def _tcb_body(agg1_ref, w2r_ref, b1r_ref, as2_ref, ad2_ref,
              xw2_ref, la_ref, ld_ref):
    h1 = [jax.nn.relu(agg1_ref[hi] + b1r_ref[hi]) for hi in range(_H)]
    for ho in range(_H):
        acc = jnp.zeros((_BLK, _LAT), jnp.float32)
        for hi in range(_H):
            acc = acc + jnp.dot(h1[hi], w2r_ref[hi, :, ho, :],
                                preferred_element_type=jnp.float32)
        xw2_ref[ho // 2, :, pl.ds((ho % 2) * _LAT, _LAT)] = acc
        dn = (((1,), (1,)), ((), ()))
        la_ref[ho] = lax.dot_general(as2_ref[ho], acc, dn,
                                     preferred_element_type=jnp.float32)
        ld_ref[ho] = lax.dot_general(ad2_ref[ho], acc, dn,
                                     preferred_element_type=jnp.float32)


def _tcb(agg1, w2r, b1r, as2, ad2):
    return pl.pallas_call(
        _tcb_body,
        grid=(_NB,),
        in_specs=[
            pl.BlockSpec((_H, _BLK, _HID), lambda b: (0, b, 0)),
            pl.BlockSpec((_H, _HID, _H, _LAT), lambda b: (0, 0, 0, 0)),
            pl.BlockSpec((_H, 1, _HID), lambda b: (0, 0, 0)),
            pl.BlockSpec((_H, 1, _LAT), lambda b: (0, 0, 0)),
            pl.BlockSpec((_H, 1, _LAT), lambda b: (0, 0, 0)),
        ],
        out_specs=[
            pl.BlockSpec((_H // 2, _BLK, 128), lambda b: (0, b, 0)),
            pl.BlockSpec((_H, 1, _BLK), lambda b: (0, 0, b)),
            pl.BlockSpec((_H, 1, _BLK), lambda b: (0, 0, b)),
        ],
        out_shape=[
            jax.ShapeDtypeStruct((_H // 2, _NP, 128), jnp.float32),
            jax.ShapeDtypeStruct((_H, 1, _NP), jnp.float32),
            jax.ShapeDtypeStruct((_H, 1, _NP), jnp.float32),
        ],
    )(agg1, w2r, b1r, as2, ad2)


def _tcc_body(agg2_ref, b2_ref, wf_ref, bf_ref, wc_ref, bc_ref,
              gxn_ref, gxc_ref, cs_ref):
    b = pl.program_id(0)
    gx = jnp.zeros((_BLK, _LAT), jnp.float32)
    for q in range(_H // 2):
        gx = gx + agg2_ref[q, :, : _LAT] + agg2_ref[q, :, _LAT:]
    gx = jax.nn.relu(gx * (1.0 / _H) + b2_ref[...])
    gxn_ref[...] = jnp.dot(gx, wf_ref[...],
                           preferred_element_type=jnp.float32) + bf_ref[...]
    gxc_ref[...] = jnp.dot(gx, wc_ref[...],
                           preferred_element_type=jnp.float32) + bc_ref[...]
    rid = lax.broadcasted_iota(jnp.int32, (_BLK, 1), 0) + b * _BLK
    gxm = jnp.where(rid < _N, gx, 0.0)

    @pl.when(b == 0)
    def _():
        cs_ref[...] = jnp.zeros_like(cs_ref)

    cs_ref[...] += gxm


def _tcc(agg2, b2, wf, bf, wc16, bc16):
    return pl.pallas_call(
        _tcc_body,
        grid=(_NB,),
        in_specs=[
            pl.BlockSpec((_H // 2, _BLK, 128), lambda b: (0, b, 0)),
            pl.BlockSpec((1, _LAT), lambda b: (0, 0)),
            pl.BlockSpec((_LAT, _LOW), lambda b: (0, 0)),
            pl.BlockSpec((1, _LOW), lambda b: (0, 0)),
            pl.BlockSpec((_LAT, 16), lambda b: (0, 0)),
            pl.BlockSpec((1, 16), lambda b: (0, 0)),
        ],
        out_specs=[
            pl.BlockSpec((_BLK, _LOW), lambda b: (b, 0)),
            pl.BlockSpec((_BLK, 16), lambda b: (b, 0)),
            pl.BlockSpec((_BLK, _LAT), lambda b: (0, 0)),
        ],
        out_shape=[
            jax.ShapeDtypeStruct((_NP, _LOW), jnp.float32),
            jax.ShapeDtypeStruct((_NP, 16), jnp.float32),
            jax.ShapeDtypeStruct((_BLK, _LAT), jnp.float32),
        ],
    )(agg2, b2, wf, bf, wc16, bc16)


def _tcm_body(x_ref, wm1_ref, bm1_ref, wm2_ref, bm2_ref, wf_ref, bf_ref,
              gxn_ref, cs_ref):
    b = pl.program_id(0)
    t = jax.nn.relu(jnp.dot(x_ref[...], wm1_ref[...],
                            preferred_element_type=jnp.float32) + bm1_ref[...])
    gx2 = jnp.dot(t, wm2_ref[...],
                  preferred_element_type=jnp.float32) + bm2_ref[...]
    gxn_ref[...] = jnp.dot(gx2, wf_ref[...],
                           preferred_element_type=jnp.float32) + bf_ref[...]
    rid = lax.broadcasted_iota(jnp.int32, (_BLK, 1), 0) + b * _BLK
    gxm = jnp.where(rid < _N, gx2, 0.0)

    @pl.when(b == 0)
    def _():
        cs_ref[...] = jnp.zeros_like(cs_ref)

    cs_ref[...] += gxm


def _tcm(xp, wm1, bm1, wm2, bm2, wf, bf):
    return pl.pallas_call(
        _tcm_body,
        grid=(_NB,),
        in_specs=[
            pl.BlockSpec((_BLK, _C), lambda b: (b, 0)),
            pl.BlockSpec((_C, _LAT), lambda b: (0, 0)),
            pl.BlockSpec((1, _LAT), lambda b: (0, 0)),
            pl.BlockSpec((_LAT, _LAT), lambda b: (0, 0)),
            pl.BlockSpec((1, _LAT), lambda b: (0, 0)),
            pl.BlockSpec((_LAT, _LOW), lambda b: (0, 0)),
            pl.BlockSpec((1, _LOW), lambda b: (0, 0)),
        ],
        out_specs=[
            pl.BlockSpec((_BLK, _LOW), lambda b: (b, 0)),
            pl.BlockSpec((_BLK, _LAT), lambda b: (0, 0)),
        ],
        out_shape=[
            jax.ShapeDtypeStruct((_NP, _LOW), jnp.float32),
            jax.ShapeDtypeStruct((_BLK, _LAT), jnp.float32),
        ],
    )(xp, wm1, bm1, wm2, bm2, wf, bf)


def kernel(x, edge_index, W1, att_src1, att_dst1, b1, W2, att_src2, att_dst2,
           b2, Wm1, bm1, Wm2, bm2, Wf, bf, Wc, bc):
    loop = jnp.arange(_N, dtype=edge_index.dtype)
    src = jnp.concatenate([edge_index[0], loop])
    dst = jnp.concatenate([edge_index[1], loop])

    noise = jax.random.normal(jax.random.key(42), x.shape, x.dtype) * _SIG
    nrm = jnp.linalg.norm(noise, axis=1, keepdims=True)
    x_aug = x + noise / jnp.maximum(nrm, 1e-12)
    xp = jnp.pad(x, ((0, _NP - _N), (0, 0)))
    xap = jnp.pad(x_aug, ((0, _NP - _N), (0, 0)))

    w1r = W1.reshape(_C, _H, _HID).transpose(1, 0, 2)
    wasT = jnp.pad(jnp.einsum('hcf,hf->hc', w1r, att_src1), ((0, 6), (0, 0)))
    wadT = jnp.pad(jnp.einsum('hcf,hf->hc', w1r, att_dst1), ((0, 6), (0, 0)))
    w2r = W2.reshape(_H, _HID, _H, _LAT)
    b1r = b1.reshape(_H, 1, _HID)
    as2 = att_src2.reshape(_H, 1, _LAT)
    ad2 = att_dst2.reshape(_H, 1, _LAT)
    b2r = b2.reshape(1, _LAT)
    bfr = bf.reshape(1, _LOW)
    wc16 = jnp.pad(Wc, ((0, 0), (0, 16 - _CLU)))
    bc16 = jnp.pad(bc, (0, 16 - _CLU)).reshape(1, 16)
    bm1r = bm1.reshape(1, _LAT)
    bm2r = bm2.reshape(1, _LAT)

    src_c, dst_c, counts = _sc_partition(src, dst)

    def enc(hp):
        xw1, as1, ad1 = _tca(hp, w1r, wasT, wadT)
        p1 = _sc_logits(as1.reshape(-1), ad1.reshape(-1), src_c, dst_c,
                        counts)
        agg1 = _sce_l1(xw1.reshape(_H * _NP, _HID), p1, src_c, dst_c, counts)
        xw2, la2, ld2 = _tcb(agg1.reshape(_H, _NP, _HID), w2r, b1r, as2, ad2)
        p2 = _sc_logits(la2.reshape(-1), ld2.reshape(-1), src_c, dst_c,
                        counts)
        agg2 = _sce_l2(xw2.reshape(_H // 2 * _NP, 128), p2, src_c, dst_c,
                       counts)
        return _tcc(agg2.reshape(_H // 2, _NP, 128), b2r, Wf, bfr, wc16, bc16)

    gxn0, gxc0, cs0 = enc(xp)
    gxn1, gxc1, cs1 = enc(xap)
    gxn2, cs2 = _tcm(xp, Wm1, bm1r, Wm2, bm2r, Wf, bfr)

    g0 = jnp.sum(cs0, axis=0) / _N
    g1 = jnp.sum(cs1, axis=0) / _N
    g2 = jnp.sum(cs2, axis=0) / _N
    fenzi = jnp.exp(jnp.dot(g0, g1) / 0.2)
    fenmu = (fenzi + jnp.exp(jnp.dot(g0, g2) / 0.2)
             + jnp.exp(jnp.dot(g1, g2) / 0.2))
    loss_graph = -jnp.log10(fenzi / fenmu)

    return (gxn0[:_N], gxn1[:_N], gxn2[:_N], loss_graph,
            gxc0[:_N, :_CLU].T, gxc1[:_N, :_CLU].T)


# speedup vs baseline: 1.3897x; 1.1469x over previous
import functools

import jax
import jax.numpy as jnp
from jax import lax
from jax.experimental import pallas as pl
from jax.experimental.pallas import tpu as pltpu
from jax.experimental.pallas import tpu_sc as plsc

_N = 10000
_E = 160000
_EP = _E + _N
_C = 128
_LAT = 64
_LOW = 32
_CLU = 10
_SIG = 0.5
_H = 10
_HID = 128
_NP = 10240
_NT = 32
_RT = _NP // _NT
_CAP = 24576
_G = 128
_S = 2000
_BLK = 256
_NB = _NP // _BLK


def _mesh():
    return plsc.VectorSubcoreMesh(core_axis_name="c", subcore_axis_name="s")


_SC_PARAMS = pltpu.CompilerParams(needs_layout_passes=False)


def _wid():
    return lax.axis_index("c") * 16 + lax.axis_index("s")


@functools.partial(
    pl.kernel,
    mesh=_mesh(),
    compiler_params=_SC_PARAMS,
    out_type=(
        jax.ShapeDtypeStruct((_NT * _CAP,), jnp.int32),
        jax.ShapeDtypeStruct((_NT * _CAP,), jnp.int32),
        jax.ShapeDtypeStruct((_NT * 16,), jnp.int32),
    ),
    scratch_types=[
        pltpu.VMEM((_S,), jnp.int32),
        pltpu.VMEM((_S,), jnp.int32),
        pltpu.VMEM((_CAP + 16,), jnp.int32),
        pltpu.VMEM((_CAP + 16,), jnp.int32),
        pltpu.VMEM((16,), jnp.int32),
        pltpu.SemaphoreType.DMA,
    ],
)
def _sc_partition(src_hbm, dst_hbm, src_c, dst_c, counts, sv, dv, sbuf, dbuf,
                  cbuf, sem):
    wid = _wid()
    base = wid * _RT
    zero = jnp.zeros((16,), jnp.int32)
    basev = jnp.full((16,), base, jnp.int32)
    lanes = lax.iota(jnp.int32, 16)
    dumpv = jnp.full((16,), _CAP, jnp.int32) + lanes

    def _zero_body(i, _):
        sbuf[pl.ds(pl.multiple_of(i * 16, 16), 16)] = zero
        dbuf[pl.ds(pl.multiple_of(i * 16, 16), 16)] = basev
        return 0

    lax.fori_loop(0, (_CAP + 16) // 16, _zero_body, 0)

    def _chunk(ci, w):
        off = pl.multiple_of(ci * _S, 8)
        pltpu.sync_copy(src_hbm.at[pl.ds(off, _S)], sv)
        pltpu.sync_copy(dst_hbm.at[pl.ds(off, _S)], dv)

        def _grp(g, w):
            o = pl.multiple_of(g * 16, 16)
            d16 = dv[pl.ds(o, 16)]
            s16 = sv[pl.ds(o, 16)]
            m = (d16 >= base) & (d16 < base + _RT)
            cs = plsc.cumsum(m.astype(jnp.int32))
            wc = jnp.minimum(w, _CAP - 16)
            pos = jnp.where(m, cs + (wc - 1), dumpv)
            plsc.store_scatter(sbuf, [pos], s16)
            plsc.store_scatter(dbuf, [pos], d16)
            return w + cs[15]

        return lax.fori_loop(0, _S // 16, _grp, w)

    w = lax.fori_loop(0, _EP // _S, _chunk, jnp.int32(0))
    cbuf[...] = jnp.full((16,), w, jnp.int32)
    obase = pl.multiple_of(wid * _CAP, 8)
    pltpu.sync_copy(sbuf.at[pl.ds(0, _CAP)], src_c.at[pl.ds(obase, _CAP)])
    pltpu.sync_copy(dbuf.at[pl.ds(0, _CAP)], dst_c.at[pl.ds(obase, _CAP)])
    pltpu.sync_copy(cbuf, counts.at[pl.ds(pl.multiple_of(wid * 16, 8), 16)])


_P = 1024


@functools.partial(
    pl.kernel,
    mesh=_mesh(),
    compiler_params=_SC_PARAMS,
    out_type=jax.ShapeDtypeStruct((_H * _NT * _CAP,), jnp.float32),
    scratch_types=[
        pltpu.VMEM((_N,), jnp.float32),
        pltpu.VMEM((_N,), jnp.float32),
        pltpu.VMEM((_P,), jnp.int32),
        pltpu.VMEM((_P,), jnp.int32),
        pltpu.VMEM((_P,), jnp.float32),
        pltpu.VMEM((16,), jnp.int32),
    ],
)
def _sc_logits(asrc_hbm, adst_hbm, srcc_hbm, dstc_hbm, cnt_hbm, p_hbm,
               at_s, at_d, si, di, po, cntv):
    wid = _wid()
    pltpu.sync_copy(cnt_hbm.at[pl.ds(pl.multiple_of(wid * 16, 8), 16)], cntv)
    cnt = jnp.minimum(cntv[pl.ds(0, 16)][0], _CAP)
    nch = (cnt + _P - 1) // _P
    lanes = lax.iota(jnp.int32, 16)

    def _head(h, _):
        toff = pl.multiple_of(h * _NP, 8)
        pltpu.sync_copy(asrc_hbm.at[pl.ds(toff, _N)], at_s)
        pltpu.sync_copy(adst_hbm.at[pl.ds(toff, _N)], at_d)

        def _chunk(ci, _):
            off = pl.multiple_of(wid * _CAP + ci * _P, 8)
            pltpu.sync_copy(srcc_hbm.at[pl.ds(off, _P)], si)
            pltpu.sync_copy(dstc_hbm.at[pl.ds(off, _P)], di)

            @plsc.parallel_loop(0, _P // 16, 1, unroll=2)
            def _grp(g):
                o = pl.multiple_of(g * 16, 16)
                s16 = si[pl.ds(o, 16)]
                d16 = di[pl.ds(o, 16)]
                av = (plsc.load_gather(at_s, [s16])
                      + plsc.load_gather(at_d, [d16]))
                av = jnp.where(av > 0, av, 0.2 * av)
                valid = (lanes + o) < (cnt - ci * _P)
                po[pl.ds(o, 16)] = jnp.where(valid, jnp.exp(av), 0.0)

            pltpu.sync_copy(
                po,
                p_hbm.at[pl.ds(pl.multiple_of(h * _NT * _CAP + off, 8), _P)])
            return 0

        lax.fori_loop(0, nch, _chunk, 0)
        return 0

    lax.fori_loop(0, _H, _head, 0)


def _make_sce(hpr, G, nb):
    npass = _H // hpr
    nvs = (128 // hpr) // 16

    @functools.partial(
        pl.kernel,
        mesh=_mesh(),
        compiler_params=_SC_PARAMS,
        out_type=jax.ShapeDtypeStruct((npass * _NP, 128), jnp.float32),
        scratch_types=[
            pltpu.VMEM((_RT, 128), jnp.float32),
            pltpu.VMEM((_RT, 16), jnp.float32),
            [pltpu.VMEM((G, 128), jnp.float32) for _ in range(nb)],
            [pltpu.VMEM((G,), jnp.int32) for _ in range(nb)],
            [pltpu.VMEM((G,), jnp.int32) for _ in range(nb)],
            [[pltpu.VMEM((G,), jnp.float32) for _ in range(hpr)]
             for _ in range(nb)],
            pltpu.VMEM((16,), jnp.int32),
            [pltpu.SemaphoreType.DMA for _ in range(nb)],
            [pltpu.SemaphoreType.DMA for _ in range(nb)],
        ],
    )
    def _sce(xw_hbm, p_hbm, srcc_hbm, dstc_hbm, cnt_hbm, agg_hbm,
             acc, accd, rowss, didxs, aidxs, pbufss, cntv, sems, isems):
        wid = _wid()
        base = wid * _RT
        pltpu.sync_copy(cnt_hbm.at[pl.ds(pl.multiple_of(wid * 16, 8), 16)], cntv)
        cnt = jnp.minimum(cntv[pl.ds(0, 16)][0], _CAP)
        nch = (cnt + G - 1) // G
        zf = jnp.zeros((16,), jnp.float32)
        lanes = lax.iota(jnp.int32, 16)

        def _pass(q, _):
            @plsc.parallel_loop(0, _RT, 1, unroll=4)
            def _zr(r):
                for c in range(8):
                    acc[r, pl.ds(16 * c, 16)] = zf
                accd[r, pl.ds(0, 16)] = zf

            hb = jnp.full((16,), q * _NP, jnp.int32)

            def _ifire(ci, b):
                off = pl.multiple_of(wid * _CAP + ci * G, 8)
                pltpu.async_copy(srcc_hbm.at[pl.ds(off, G)], aidxs[b],
                                 isems[b])
                pltpu.async_copy(dstc_hbm.at[pl.ds(off, G)], didxs[b],
                                 isems[b])
                for sub in range(hpr):
                    poff = pl.multiple_of(
                        (q * hpr + sub) * _NT * _CAP + off, 8)
                    pltpu.async_copy(p_hbm.at[pl.ds(poff, G)],
                                     pbufss[b][sub], isems[b])

            def _idrain(b):
                pltpu.make_async_copy(
                    srcc_hbm.at[pl.ds(0, G)], aidxs[b], isems[b]).wait()
                pltpu.make_async_copy(
                    srcc_hbm.at[pl.ds(0, G)], didxs[b], isems[b]).wait()
                for sub in range(hpr):
                    pltpu.make_async_copy(
                        p_hbm.at[pl.ds(0, G)], pbufss[b][sub],
                        isems[b]).wait()

            def _gfire(b):
                @plsc.parallel_loop(0, G // 16, 1, unroll=2)
                def _adj(g):
                    o = pl.multiple_of(g * 16, 16)
                    aidxs[b][pl.ds(o, 16)] = aidxs[b][pl.ds(o, 16)] + hb

                pltpu.async_copy(xw_hbm.at[aidxs[b]], rowss[b], sems[b])

            def _stage(ci, b):
                _ifire(ci, b)
                _idrain(b)
                _gfire(b)

            def _proc(b):
                pltpu.make_async_copy(
                    xw_hbm.at[pl.ds(0, G)], rowss[b], sems[b]).wait()

                @plsc.parallel_loop(0, G // 16, 1, unroll=2)
                def _egrp(g):
                    o = pl.multiple_of(g * 16, 16)
                    dv16 = jnp.clip(didxs[b][pl.ds(o, 16)] - base, 0, _RT - 1)
                    pv16s = [pbufss[b][sub][pl.ds(o, 16)]
                             for sub in range(hpr)]
                    for lane in range(16):
                        dl = dv16[lane]
                        pvs = [jnp.full((16,), pv16s[sub][lane], jnp.float32)
                               for sub in range(hpr)]
                        for c in range(8):
                            plsc.addupdate(
                                acc.at[dl, pl.ds(16 * c, 16)],
                                pvs[c // nvs] * rowss[b][o + lane,
                                                         pl.ds(16 * c, 16)])
                        if hpr == 1:
                            pvc = pvs[0]
                        else:
                            pvc = jnp.where(lanes < 8, pvs[0], pvs[1])
                        plsc.addupdate(accd.at[dl], pvc)

            if nb == 1:
                def _chunk(ci, _):
                    _stage(ci, 0)
                    _proc(0)
                    return 0

                lax.fori_loop(0, nch, _chunk, 0)
            else:
                @pl.when(nch > 0)
                def _():
                    _ifire(0, 0)

                def _outer(co, _):
                    for b in range(2):
                        ci = co * 2 + b

                        @pl.when(ci < nch)
                        def _():
                            _idrain(b)
                            _gfire(b)

                            @pl.when(ci > 0)
                            def _():
                                _proc(1 - b)

                            @pl.when(ci + 1 < nch)
                            def _():
                                _ifire(ci + 1, 1 - b)
                    return 0

                lax.fori_loop(0, (nch + 1) // 2, _outer, 0)

                @pl.when((nch > 0) & (((nch - 1) & 1) == 0))
                def _():
                    _proc(0)

                @pl.when((nch > 0) & (((nch - 1) & 1) == 1))
                def _():
                    _proc(1)

            @plsc.parallel_loop(0, _RT, 1, unroll=4)
            def _nrm(r):
                invv = 1.0 / (accd[r, pl.ds(0, 16)] + 1e-16)
                ivs = [jnp.full((16,), invv[8 * sub], jnp.float32)
                       for sub in range(hpr)]
                for c in range(8):
                    acc[r, pl.ds(16 * c, 16)] = (acc[r, pl.ds(16 * c, 16)]
                                                 * ivs[c // nvs])
            pltpu.sync_copy(
                acc,
                agg_hbm.at[pl.ds(pl.multiple_of(q * _NP + base, 8), _RT)])
            return 0

        lax.fori_loop(0, npass, _pass, 0)

    return _sce


_sce_l1 = _make_sce(1, 128, 2)
_sce_l2 = _make_sce(2, 128, 2)


def _tca_body(x_ref, w1r_ref, wasT_ref, wadT_ref, xw_ref, as_ref, ad_ref):
    xb = x_ref[...]
    for h in range(_H):
        xw_ref[h] = jnp.dot(xb, w1r_ref[h], preferred_element_type=jnp.float32)
    dn = (((1,), (1,)), ((), ()))
    as_ref[...] = lax.dot_general(wasT_ref[...], xb, dn,
                                  preferred_element_type=jnp.float32)
    ad_ref[...] = lax.dot_general(wadT_ref[...], xb, dn,
                                  preferred_element_type=jnp.float32)


def _tca(xp, w1r, wasT, wadT):
    return pl.pallas_call(
        _tca_body,
        grid=(_NB,),
        in_specs=[
            pl.BlockSpec((_BLK, _C), lambda b: (b, 0)),
            pl.BlockSpec((_H, _C, _HID), lambda b: (0, 0, 0)),
            pl.BlockSpec((16, _C), lambda b: (0, 0)),
            pl.BlockSpec((16, _C), lambda b: (0, 0)),
        ],
        out_specs=[
            pl.BlockSpec((_H, _BLK, _HID), lambda b: (0, b, 0)),
            pl.BlockSpec((16, _BLK), lambda b: (0, b)),
            pl.BlockSpec((16, _BLK), lambda b: (0, b)),
        ],
        out_shape=[
            jax.ShapeDtypeStruct((_H, _NP, _HID), jnp.float32),
            jax.ShapeDtypeStruct((16, _NP), jnp.float32),
            jax.ShapeDtypeStruct((16, _NP), jnp.float32),
        ],
    )(xp, w1r, wasT, wadT)


def _tcb_body(agg1_ref, w2r_ref, b1r_ref, as2_ref, ad2_ref,
              xw2_ref, la_ref, ld_ref):
    h1 = [jax.nn.relu(agg1_ref[hi] + b1r_ref[hi]) for hi in range(_H)]
    for ho in range(_H):
        acc = jnp.zeros((_BLK, _LAT), jnp.float32)
        for hi in range(_H):
            acc = acc + jnp.dot(h1[hi], w2r_ref[hi, :, ho, :],
                                preferred_element_type=jnp.float32)
        xw2_ref[ho // 2, :, pl.ds((ho % 2) * _LAT, _LAT)] = acc
        dn = (((1,), (1,)), ((), ()))
        la_ref[ho] = lax.dot_general(as2_ref[ho], acc, dn,
                                     preferred_element_type=jnp.float32)
        ld_ref[ho] = lax.dot_general(ad2_ref[ho], acc, dn,
                                     preferred_element_type=jnp.float32)


def _tcb(agg1, w2r, b1r, as2, ad2):
    return pl.pallas_call(
        _tcb_body,
        grid=(_NB,),
        in_specs=[
            pl.BlockSpec((_H, _BLK, _HID), lambda b: (0, b, 0)),
            pl.BlockSpec((_H, _HID, _H, _LAT), lambda b: (0, 0, 0, 0)),
            pl.BlockSpec((_H, 1, _HID), lambda b: (0, 0, 0)),
            pl.BlockSpec((_H, 1, _LAT), lambda b: (0, 0, 0)),
            pl.BlockSpec((_H, 1, _LAT), lambda b: (0, 0, 0)),
        ],
        out_specs=[
            pl.BlockSpec((_H // 2, _BLK, 128), lambda b: (0, b, 0)),
            pl.BlockSpec((_H, 1, _BLK), lambda b: (0, 0, b)),
            pl.BlockSpec((_H, 1, _BLK), lambda b: (0, 0, b)),
        ],
        out_shape=[
            jax.ShapeDtypeStruct((_H // 2, _NP, 128), jnp.float32),
            jax.ShapeDtypeStruct((_H, 1, _NP), jnp.float32),
            jax.ShapeDtypeStruct((_H, 1, _NP), jnp.float32),
        ],
    )(agg1, w2r, b1r, as2, ad2)


def _tcc_body(agg2_ref, b2_ref, wf_ref, bf_ref, wc_ref, bc_ref,
              gxn_ref, gxc_ref, cs_ref):
    b = pl.program_id(0)
    gx = jnp.zeros((_BLK, _LAT), jnp.float32)
    for q in range(_H // 2):
        gx = gx + agg2_ref[q, :, : _LAT] + agg2_ref[q, :, _LAT:]
    gx = jax.nn.relu(gx * (1.0 / _H) + b2_ref[...])
    gxn_ref[...] = jnp.dot(gx, wf_ref[...],
                           preferred_element_type=jnp.float32) + bf_ref[...]
    gxc_ref[...] = jnp.dot(gx, wc_ref[...],
                           preferred_element_type=jnp.float32) + bc_ref[...]
    rid = lax.broadcasted_iota(jnp.int32, (_BLK, 1), 0) + b * _BLK
    gxm = jnp.where(rid < _N, gx, 0.0)

    @pl.when(b == 0)
    def _():
        cs_ref[...] = jnp.zeros_like(cs_ref)

    cs_ref[...] += gxm


def _tcc(agg2, b2, wf, bf, wc16, bc16):
    return pl.pallas_call(
        _tcc_body,
        grid=(_NB,),
        in_specs=[
            pl.BlockSpec((_H // 2, _BLK, 128), lambda b: (0, b, 0)),
            pl.BlockSpec((1, _LAT), lambda b: (0, 0)),
            pl.BlockSpec((_LAT, _LOW), lambda b: (0, 0)),
            pl.BlockSpec((1, _LOW), lambda b: (0, 0)),
            pl.BlockSpec((_LAT, 16), lambda b: (0, 0)),
            pl.BlockSpec((1, 16), lambda b: (0, 0)),
        ],
        out_specs=[
            pl.BlockSpec((_BLK, _LOW), lambda b: (b, 0)),
            pl.BlockSpec((_BLK, 16), lambda b: (b, 0)),
            pl.BlockSpec((_BLK, _LAT), lambda b: (0, 0)),
        ],
        out_shape=[
            jax.ShapeDtypeStruct((_NP, _LOW), jnp.float32),
            jax.ShapeDtypeStruct((_NP, 16), jnp.float32),
            jax.ShapeDtypeStruct((_BLK, _LAT), jnp.float32),
        ],
    )(agg2, b2, wf, bf, wc16, bc16)


def _tcm_body(x_ref, wm1_ref, bm1_ref, wm2_ref, bm2_ref, wf_ref, bf_ref,
              gxn_ref, cs_ref):
    b = pl.program_id(0)
    t = jax.nn.relu(jnp.dot(x_ref[...], wm1_ref[...],
                            preferred_element_type=jnp.float32) + bm1_ref[...])
    gx2 = jnp.dot(t, wm2_ref[...],
                  preferred_element_type=jnp.float32) + bm2_ref[...]
    gxn_ref[...] = jnp.dot(gx2, wf_ref[...],
                           preferred_element_type=jnp.float32) + bf_ref[...]
    rid = lax.broadcasted_iota(jnp.int32, (_BLK, 1), 0) + b * _BLK
    gxm = jnp.where(rid < _N, gx2, 0.0)

    @pl.when(b == 0)
    def _():
        cs_ref[...] = jnp.zeros_like(cs_ref)

    cs_ref[...] += gxm


def _tcm(xp, wm1, bm1, wm2, bm2, wf, bf):
    return pl.pallas_call(
        _tcm_body,
        grid=(_NB,),
        in_specs=[
            pl.BlockSpec((_BLK, _C), lambda b: (b, 0)),
            pl.BlockSpec((_C, _LAT), lambda b: (0, 0)),
            pl.BlockSpec((1, _LAT), lambda b: (0, 0)),
            pl.BlockSpec((_LAT, _LAT), lambda b: (0, 0)),
            pl.BlockSpec((1, _LAT), lambda b: (0, 0)),
            pl.BlockSpec((_LAT, _LOW), lambda b: (0, 0)),
            pl.BlockSpec((1, _LOW), lambda b: (0, 0)),
        ],
        out_specs=[
            pl.BlockSpec((_BLK, _LOW), lambda b: (b, 0)),
            pl.BlockSpec((_BLK, _LAT), lambda b: (0, 0)),
        ],
        out_shape=[
            jax.ShapeDtypeStruct((_NP, _LOW), jnp.float32),
            jax.ShapeDtypeStruct((_BLK, _LAT), jnp.float32),
        ],
    )(xp, wm1, bm1, wm2, bm2, wf, bf)


def kernel(x, edge_index, W1, att_src1, att_dst1, b1, W2, att_src2, att_dst2,
           b2, Wm1, bm1, Wm2, bm2, Wf, bf, Wc, bc):
    loop = jnp.arange(_N, dtype=edge_index.dtype)
    src = jnp.concatenate([edge_index[0], loop])
    dst = jnp.concatenate([edge_index[1], loop])

    noise = jax.random.normal(jax.random.key(42), x.shape, x.dtype) * _SIG
    nrm = jnp.linalg.norm(noise, axis=1, keepdims=True)
    x_aug = x + noise / jnp.maximum(nrm, 1e-12)
    xp = jnp.pad(x, ((0, _NP - _N), (0, 0)))
    xap = jnp.pad(x_aug, ((0, _NP - _N), (0, 0)))

    w1r = W1.reshape(_C, _H, _HID).transpose(1, 0, 2)
    wasT = jnp.pad(jnp.einsum('hcf,hf->hc', w1r, att_src1), ((0, 6), (0, 0)))
    wadT = jnp.pad(jnp.einsum('hcf,hf->hc', w1r, att_dst1), ((0, 6), (0, 0)))
    w2r = W2.reshape(_H, _HID, _H, _LAT)
    b1r = b1.reshape(_H, 1, _HID)
    as2 = att_src2.reshape(_H, 1, _LAT)
    ad2 = att_dst2.reshape(_H, 1, _LAT)
    b2r = b2.reshape(1, _LAT)
    bfr = bf.reshape(1, _LOW)
    wc16 = jnp.pad(Wc, ((0, 0), (0, 16 - _CLU)))
    bc16 = jnp.pad(bc, (0, 16 - _CLU)).reshape(1, 16)
    bm1r = bm1.reshape(1, _LAT)
    bm2r = bm2.reshape(1, _LAT)

    src_c, dst_c, counts = _sc_partition(src, dst)

    def enc(hp):
        xw1, as1, ad1 = _tca(hp, w1r, wasT, wadT)
        p1 = _sc_logits(as1.reshape(-1), ad1.reshape(-1), src_c, dst_c,
                        counts)
        agg1 = _sce_l1(xw1.reshape(_H * _NP, _HID), p1, src_c, dst_c, counts)
        xw2, la2, ld2 = _tcb(agg1.reshape(_H, _NP, _HID), w2r, b1r, as2, ad2)
        p2 = _sc_logits(la2.reshape(-1), ld2.reshape(-1), src_c, dst_c,
                        counts)
        agg2 = _sce_l2(xw2.reshape(_H // 2 * _NP, 128), p2, src_c, dst_c,
                       counts)
        return _tcc(agg2.reshape(_H // 2, _NP, 128), b2r, Wf, bfr, wc16, bc16)

    gxn0, gxc0, cs0 = enc(xp)
    gxn1, gxc1, cs1 = enc(xap)
    gxn2, cs2 = _tcm(xp, Wm1, bm1r, Wm2, bm2r, Wf, bfr)

    g0 = jnp.sum(cs0, axis=0) / _N
    g1 = jnp.sum(cs1, axis=0) / _N
    g2 = jnp.sum(cs2, axis=0) / _N
    fenzi = jnp.exp(jnp.dot(g0, g1) / 0.2)
    fenmu = (fenzi + jnp.exp(jnp.dot(g0, g2) / 0.2)
             + jnp.exp(jnp.dot(g1, g2) / 0.2))
    loss_graph = -jnp.log10(fenzi / fenmu)

    return (gxn0[:_N], gxn1[:_N], gxn2[:_N], loss_graph,
            gxc0[:_N, :_CLU].T, gxc1[:_N, :_CLU].T)
